# baseline (speedup 1.0000x reference)
import math
from collections import Counter, defaultdict

import jax
import jax.numpy as jnp
from jax import lax
from jax.experimental import pallas as pl
from jax.experimental.pallas import tpu as pltpu
from jax.experimental.pallas import tpu_sc as plsc

_MAX_L = 3


def _build_terms():
    l_list = []
    for l in range(_MAX_L + 1):
        for lx in range(l, -1, -1):
            for ly in range(l - lx, -1, -1):
                l_list.append((lx, ly, l - lx - ly))
    l_index = {v: i for i, v in enumerate(l_list)}

    def mnom(k, m):
        return math.factorial(k) / (
            math.factorial(m[0]) * math.factorial(m[1]) * math.factorial(m[2]))

    def comps(k):
        return [(mx, my, k - mx - my) for mx in range(k + 1) for my in range(k + 1 - mx)]

    terms = defaultdict(float)
    n = 0
    for l in range(1, _MAX_L + 1):
        for v in comps(l):
            terms[(1 + n, (l_index[v], l_index[v]))] += mnom(l, v)
        n += 1
    n2 = n
    gidx = 0
    for k12 in range(_MAX_L + 1):
        for k13 in range(k12 + 1):
            for k23 in range(k13 + 1):
                l1, l2, l3 = k12 + k13, k12 + k23, k13 + k23
                if min(l1, l2, l3) < 1 or max(l1, l2, l3) > _MAX_L:
                    continue
                for m12 in comps(k12):
                    for m13 in comps(k13):
                        for m23 in comps(k23):
                            v1 = tuple(a + b for a, b in zip(m12, m13))
                            v2 = tuple(a + b for a, b in zip(m12, m23))
                            v3 = tuple(a + b for a, b in zip(m13, m23))
                            ids = tuple(sorted(
                                (l_index[v1], l_index[v2], l_index[v3])))
                            terms[(1 + n2 + gidx, ids)] += (
                                mnom(k12, m12) * mnom(k13, m13) * mnom(k23, m23))
                gidx += 1
    merged = sorted(terms.items())
    n_slots = 1 + n2 + gidx

    sq_pairs = {ids for (_, ids), _ in merged if len(ids) == 2}
    cubics = [(o, ids, pf) for (o, ids), pf in merged if len(ids) == 3]
    cand_count = Counter()
    for _, ids, _ in cubics:
        a, b, c = ids
        for p in {(a, b), (a, c), (b, c)}:
            cand_count[p] += 1
    cache = set(sq_pairs)
    plan3 = []
    for o, ids, pf in cubics:
        a, b, c = ids
        cands = [(a, b), (a, c), (b, c)]
        hit = [p for p in cands if p in cache]
        if hit:
            p = hit[0]
        else:
            p = max(cands, key=lambda q: cand_count[q])
            cache.add(p)
        rest = list(ids)
        for e in p:
            rest.remove(e)
        plan3.append((o, p, rest[0], pf))
    plan2 = [(o, ids, pf) for (o, ids), pf in merged if len(ids) == 2]
    pairs = sorted(cache)

    uses2 = defaultdict(list)
    for o, p, pf in plan2:
        uses2[p].append((o, pf))
    uses3 = defaultdict(list)
    for o, p, c, pf in plan3:
        uses3[(p, o)].append((c, pf))
    sched = []
    for p in pairs:
        groups3 = [(o, cl) for (q, o), cl in sorted(uses3.items()) if q == p]
        sched.append((p, uses2.get(p, []), groups3))
    return sched, n_slots


_SCHED, _N_SLOTS = _build_terms()

_N, _R, _A, _C = 10000, 8, 20, 4
_LANES = 128
_NT = -(-_N // _LANES)
_NPAD = _NT * _LANES
_SLAB = _C * _LANES
_UNITS = _R * _NT
_NW = 32
_VCHUNKS = _SLAB // 16


_W = 5
_BPR = 16
_BLOCKS = _R * _BPR
_BW = _W * _SLAB


def _sym_body(in_hbm, out_hbm, in_buf, out_buf):
    wid = lax.axis_index("s") * 2 + lax.axis_index("c")

    def block_body(b, _):
        r = b // _BPR
        t0 = lax.min(lax.rem(b, _BPR) * _W, _NT - _W)
        pltpu.sync_copy(
            in_hbm.at[pl.ds((r * _NT + t0) * _A * _SLAB, _W * _A * _SLAB)],
            in_buf)

        @plsc.parallel_loop(0, _W * _VCHUNKS, unroll=2)
        def chunk_body(j):
            ti = j // _VCHUNKS
            base = (j % _VCHUNKS) * 16
            src = ti * _A * _SLAB + base
            xs = [in_buf[pl.ds(src + a * _SLAB, 16)] for a in range(_A)]
            acc = [None] * _N_SLOTS
            acc[0] = xs[0]
            for p, u2, g3 in _SCHED:
                prod = xs[p[0]] * xs[p[1]]
                for o, pf in u2:
                    t2 = prod if pf == 1.0 else prod * pf
                    acc[o] = t2 if acc[o] is None else acc[o] + t2
                for o, clist in g3:
                    inner = None
                    for c, pf in clist:
                        v = xs[c] if pf == 1.0 else xs[c] * pf
                        inner = v if inner is None else inner + v
                    t3 = prod * inner
                    acc[o] = t3 if acc[o] is None else acc[o] + t3
            dst = ti * _SLAB + base
            for s in range(_N_SLOTS):
                out_buf[pl.ds(s * _BW + dst, 16)] = acc[s]

        for s in range(_N_SLOTS):
            pltpu.sync_copy(
                out_buf.at[pl.ds(s * _BW, _BW)],
                out_hbm.at[pl.ds(((r * _N_SLOTS + s) * _NT + t0) * _SLAB, _BW)])
        return 0

    lax.fori_loop(wid * (_BLOCKS // _NW), (wid + 1) * (_BLOCKS // _NW),
                  block_body, 0)


@jax.jit
def kernel(node_attr):
    n, r, a, c = node_attr.shape
    x = node_attr.transpose(1, 0, 2, 3)
    x = jnp.pad(x, ((0, 0), (0, _NPAD - n), (0, 0), (0, 0)))
    x = x.reshape(r, _NT, _LANES, a, c)
    x = x.transpose(0, 1, 3, 4, 2).reshape(-1)
    run = pl.kernel(
        _sym_body,
        out_type=jax.ShapeDtypeStruct((_R * _N_SLOTS * _NT * _SLAB,),
                                      jnp.float32),
        mesh=plsc.VectorSubcoreMesh(core_axis_name="c", subcore_axis_name="s"),
        compiler_params=pltpu.CompilerParams(needs_layout_passes=False),
        scratch_types=[
            pltpu.VMEM((_W * _A * _SLAB,), jnp.float32),
            pltpu.VMEM((_N_SLOTS * _BW,), jnp.float32),
        ],
    )
    out = run(x)
    out = out.reshape(r, _N_SLOTS, _NT, c, _LANES)
    out = out.transpose(2, 4, 0, 1, 3).reshape(_NPAD, r, _N_SLOTS, c)
    return out[:n]


# speedup vs baseline: 1.3778x; 1.3778x over previous
import math
from collections import Counter, defaultdict

import jax
import jax.numpy as jnp
from jax import lax
from jax.experimental import pallas as pl
from jax.experimental.pallas import tpu as pltpu
from jax.experimental.pallas import tpu_sc as plsc

_MAX_L = 3


def _build_terms():
    l_list = []
    for l in range(_MAX_L + 1):
        for lx in range(l, -1, -1):
            for ly in range(l - lx, -1, -1):
                l_list.append((lx, ly, l - lx - ly))
    l_index = {v: i for i, v in enumerate(l_list)}

    def mnom(k, m):
        return math.factorial(k) / (
            math.factorial(m[0]) * math.factorial(m[1]) * math.factorial(m[2]))

    def comps(k):
        return [(mx, my, k - mx - my) for mx in range(k + 1) for my in range(k + 1 - mx)]

    terms = defaultdict(float)
    n = 0
    for l in range(1, _MAX_L + 1):
        for v in comps(l):
            terms[(1 + n, (l_index[v], l_index[v]))] += mnom(l, v)
        n += 1
    n2 = n
    gidx = 0
    for k12 in range(_MAX_L + 1):
        for k13 in range(k12 + 1):
            for k23 in range(k13 + 1):
                l1, l2, l3 = k12 + k13, k12 + k23, k13 + k23
                if min(l1, l2, l3) < 1 or max(l1, l2, l3) > _MAX_L:
                    continue
                for m12 in comps(k12):
                    for m13 in comps(k13):
                        for m23 in comps(k23):
                            v1 = tuple(a + b for a, b in zip(m12, m13))
                            v2 = tuple(a + b for a, b in zip(m12, m23))
                            v3 = tuple(a + b for a, b in zip(m13, m23))
                            ids = tuple(sorted(
                                (l_index[v1], l_index[v2], l_index[v3])))
                            terms[(1 + n2 + gidx, ids)] += (
                                mnom(k12, m12) * mnom(k13, m13) * mnom(k23, m23))
                gidx += 1
    merged = sorted(terms.items())
    n_slots = 1 + n2 + gidx

    sq_pairs = {ids for (_, ids), _ in merged if len(ids) == 2}
    cubics = [(o, ids, pf) for (o, ids), pf in merged if len(ids) == 3]
    cand_count = Counter()
    for _, ids, _ in cubics:
        a, b, c = ids
        for p in {(a, b), (a, c), (b, c)}:
            cand_count[p] += 1
    cache = set(sq_pairs)
    plan3 = []
    for o, ids, pf in cubics:
        a, b, c = ids
        cands = [(a, b), (a, c), (b, c)]
        hit = [p for p in cands if p in cache]
        if hit:
            p = hit[0]
        else:
            p = max(cands, key=lambda q: cand_count[q])
            cache.add(p)
        rest = list(ids)
        for e in p:
            rest.remove(e)
        plan3.append((o, p, rest[0], pf))
    plan2 = [(o, ids, pf) for (o, ids), pf in merged if len(ids) == 2]
    pairs = sorted(cache)

    uses2 = defaultdict(list)
    for o, p, pf in plan2:
        uses2[p].append((o, pf))
    uses3 = defaultdict(list)
    for o, p, c, pf in plan3:
        uses3[(p, o)].append((c, pf))
    sched = []
    for p in pairs:
        groups3 = [(o, cl) for (q, o), cl in sorted(uses3.items()) if q == p]
        sched.append((p, uses2.get(p, []), groups3))
    return sched, n_slots


_SCHED, _N_SLOTS = _build_terms()

_N, _R, _A, _C = 10000, 8, 20, 4
_LANES = 128
_NTF = _N // _LANES
_TAIL = _N - _NTF * _LANES
_NT = _NTF + 1
_NPAD = _NT * _LANES
_SLAB = _C * _LANES
_NW = 32

_W = 6
_BPR = _NTF // _W
_MB = _R * _BPR
_BN = _W * _LANES
_BS = _W * _SLAB


def _eval_chunk(read):
    xs = [read(a) for a in range(_A)]
    acc = [None] * _N_SLOTS
    acc[0] = xs[0]
    for p, u2, g3 in _SCHED:
        prod = xs[p[0]] * xs[p[1]]
        for o, pf in u2:
            t2 = prod if pf == 1.0 else prod * pf
            acc[o] = t2 if acc[o] is None else acc[o] + t2
        for o, clist in g3:
            inner = None
            for c, pf in clist:
                v = xs[c] if pf == 1.0 else xs[c] * pf
                inner = v if inner is None else inner + v
            t3 = prod * inner
            acc[o] = t3 if acc[o] is None else acc[o] + t3
    return acc


def _sym_body(in_hbm, out_hbm, in_buf, out_buf, tin_buf, tout_buf):
    wid = lax.axis_index("s") * 2 + lax.axis_index("c")

    def block_body(b, _):
        r = b // _BPR
        t0 = lax.rem(b, _BPR) * _W
        pltpu.sync_copy(
            in_hbm.at[r, :, :, pl.ds(t0 * _LANES, _BN)], in_buf)

        @plsc.parallel_loop(0, _W * 8)
        def chunk_body(q):
            for c in range(_C):
                acc = _eval_chunk(lambda a, c=c: in_buf[a, c, pl.ds(q * 16, 16)])
                dst = (q // 8) * _SLAB + c * _LANES + lax.rem(q, 8) * 16
                for s in range(_N_SLOTS):
                    out_buf[s, pl.ds(dst, 16)] = acc[s]

        pltpu.sync_copy(
            out_buf, out_hbm.at[r, :, pl.ds(t0 * _SLAB, _BS)])
        return 0

    def tail_body(r, _):
        pltpu.sync_copy(
            in_hbm.at[r, :, :, pl.ds(_NTF * _LANES, _TAIL)], tin_buf)
        for c in range(_C):
            acc = _eval_chunk(lambda a, c=c: tin_buf[a, c, pl.ds(0, 16)])
            for s in range(_N_SLOTS):
                tout_buf[s, pl.ds(c * _LANES, 16)] = acc[s]
        pltpu.sync_copy(
            tout_buf, out_hbm.at[r, :, pl.ds(_NTF * _SLAB, _SLAB)])
        return 0

    lax.fori_loop(wid * _MB // _NW, (wid + 1) * _MB // _NW, block_body, 0)
    lax.fori_loop(wid * _R // _NW, (wid + 1) * _R // _NW, tail_body, 0)


@jax.jit
def kernel(node_attr):
    n, r, a, c = node_attr.shape
    xt = node_attr.transpose(1, 2, 3, 0)
    run = pl.kernel(
        _sym_body,
        out_type=jax.ShapeDtypeStruct((_R, _N_SLOTS, _NT * _SLAB), jnp.float32),
        mesh=plsc.VectorSubcoreMesh(core_axis_name="c", subcore_axis_name="s"),
        compiler_params=pltpu.CompilerParams(needs_layout_passes=False),
        scratch_types=[
            pltpu.VMEM((_A, _C, _BN), jnp.float32),
            pltpu.VMEM((_N_SLOTS, _BS), jnp.float32),
            pltpu.VMEM((_A, _C, _TAIL), jnp.float32),
            pltpu.VMEM((_N_SLOTS, _SLAB), jnp.float32),
        ],
    )
    out = run(xt)
    out = out.reshape(r, _N_SLOTS, _NT, _C, _LANES)
    out = out.transpose(2, 4, 0, 1, 3).reshape(_NPAD, r, _N_SLOTS, _C)
    return out[:n]
